# baseline (speedup 1.0000x reference)
import functools

import jax
import jax.numpy as jnp
from jax import lax
from jax.experimental import pallas as pl
from jax.experimental.pallas import tpu as pltpu
from jax.experimental.pallas import tpu_sc as plsc

_B, _N, _T, _K, _D, _V, _TOK = 8, 256, 8, 7, 256, 7, 64
_KD = _K * _D
_VP = 8
_NB = 64

_NPAIR = _B * _N
_NROW = _NPAIR * _T
_NW = 32
_PPW = _NPAIR // _NW
_HALF = _PPW // 2
_RPH = _HALF * _T
_L = 16
_SL = _KD // _L
_CH = 16
_DUMP = _HALF


def _sc_reduce_kernel(emb_hbm, mask_hbm, out_hbm,
                      maskv, idxv, dstv, acc, row0, row1, sem0, sem1):
    wid = lax.axis_index("s") * 2 + lax.axis_index("c")
    pair_base = wid * _PPW
    iota = lax.iota(jnp.int32, _L)
    zero16 = jnp.zeros((_L,), jnp.float32)

    pltpu.sync_copy(mask_hbm.at[pl.ds(pair_base * _T, _PPW * _T)], maskv)

    def _half_body(half, hcarry):
        row_base = (pair_base + half * _HALF) * _T

        def _zero_body(j, carry):
            for s in range(8):
                acc[pl.ds(j * 8 * _L + s * _L, _L)] = zero16
            return carry
        lax.fori_loop(0, (_HALF + 1) * _SL // 8, _zero_body, 0)

        for i in range(_RPH // _L):
            idxv[pl.ds(i * _L, _L)] = jnp.zeros((_L,), jnp.int32)
            dstv[pl.ds(i * _L, _L)] = jnp.full((_L,), _DUMP, jnp.int32)

        cntv = jnp.zeros((_L,), jnp.int32)
        for i in range(_RPH // _L):
            mv = maskv[pl.ds(half * _RPH + i * _L, _L)] != 0
            mi = mv.astype(jnp.int32)
            pos = plsc.cumsum(mi) - 1 + cntv
            rid = iota + (row_base + i * _L)
            dpair = lax.shift_right_logical(iota + i * _L, 3)
            plsc.store_scatter(idxv, [pos], rid, mask=mv)
            plsc.store_scatter(dstv, [pos], dpair, mask=mv)
            cntv = cntv + plsc.all_reduce_population_count(mv)

        cnt = lax.reduce_max(cntv, axes=(0,))
        n_ch = (cnt + _CH - 1) // _CH

        def _fire(c, buf, sem):
            pltpu.async_copy(emb_hbm.at[idxv.at[pl.ds(c * _CH, _CH)]], buf, sem)

        @pl.when(n_ch > 0)
        def _prime0():
            _fire(0, row0, sem0)

        @pl.when(n_ch > 1)
        def _prime1():
            _fire(1, row1, sem1)

        def _accumulate(c, buf):
            for r4 in range(_CH // 4):
                rows = [r4 * 4 + k for k in range(4)]
                bases = []
                for r in rows:
                    drow = plsc.load_gather(
                        dstv, [jnp.full((_L,), 0, jnp.int32) + (c * _CH + r)])
                    bases.append(drow * _KD + iota)

                def _slice_body(j, carry):
                    off = j * (4 * _L)
                    vals = []
                    for s4 in range(4):
                        for k, r in enumerate(rows):
                            vals.append((k, off + s4 * _L,
                                         buf[r, pl.ds(off + s4 * _L, _L)]))
                    for k, o, v in vals:
                        plsc.addupdate_scatter(acc, [bases[k] + o], v)
                    return carry
                lax.fori_loop(0, _SL // 4, _slice_body, 0)

        def _chunk_body(c2, carry):
            for par, (buf, sem) in enumerate(((row0, sem0), (row1, sem1))):
                c = c2 * 2 + par

                @pl.when(c < n_ch)
                def _do():
                    pltpu.make_async_copy(
                        emb_hbm.at[idxv.at[pl.ds(c * _CH, _CH)]], buf, sem).wait()
                    _accumulate(c, buf)

                    @pl.when(c + 2 < n_ch)
                    def _next():
                        _fire(c + 2, buf, sem)
            return carry

        lax.fori_loop(0, (n_ch + 1) // 2, _chunk_body, 0)

        pltpu.sync_copy(acc.at[pl.ds(0, _HALF * _KD)],
                        out_hbm.at[pl.ds(row_base // _T * _KD, _HALF * _KD)])
        return hcarry

    lax.fori_loop(0, 2, _half_body, 0)


def _sc_reduce(emb_rows, mask_i32):
    mesh = plsc.VectorSubcoreMesh(core_axis_name="c", subcore_axis_name="s",
                                  num_cores=2, num_subcores=16)
    f = pl.kernel(
        _sc_reduce_kernel,
        out_type=jax.ShapeDtypeStruct((_NPAIR * _KD,), jnp.float32),
        mesh=mesh,
        scratch_types=[
            pltpu.VMEM((_PPW * _T,), jnp.int32),
            pltpu.VMEM((_RPH,), jnp.int32),
            pltpu.VMEM((_RPH,), jnp.int32),
            pltpu.VMEM(((_HALF + 1) * _KD,), jnp.float32),
            pltpu.VMEM((_CH, _KD), jnp.float32),
            pltpu.VMEM((_CH, _KD), jnp.float32),
            pltpu.SemaphoreType.DMA,
            pltpu.SemaphoreType.DMA,
        ],
        compiler_params=pltpu.CompilerParams(needs_layout_passes=False),
    )
    return f(emb_rows, mask_i32)


def _proj_kernel(acc_ref, vis_ref, m_ref, wemb_ref, wvis_ref, bias_ref, out_ref):
    m = m_ref[...]
    s = jnp.sum(m, axis=1, keepdims=True)
    scale = jnp.where(s > 0.0, 1.0 / jnp.maximum(s, 1.0), 0.0)
    w = m * scale

    ew = acc_ref[...] * scale
    vis = vis_ref[...]
    vw = jnp.sum(vis * w[:, :, None], axis=1)

    out = jax.lax.dot_general(ew, wemb_ref[...], (((1,), (0,)), ((), ())),
                              preferred_element_type=jnp.float32)
    out = out + jax.lax.dot_general(vw, wvis_ref[...], (((1,), (0,)), ((), ())),
                                    preferred_element_type=jnp.float32)
    any_m = (s > 0.0).astype(jnp.float32)
    out_ref[...] = out + any_m * bias_ref[...]


def _tc_proj(acc2, vis3, m2, wemb, wvis, bias):
    grid = (_NPAIR // _NB,)
    return pl.pallas_call(
        _proj_kernel,
        grid=grid,
        in_specs=[
            pl.BlockSpec((_NB, _KD), lambda j: (j, 0)),
            pl.BlockSpec((_NB, _T, _VP), lambda j: (j, 0, 0)),
            pl.BlockSpec((_NB, _T), lambda j: (j, 0)),
            pl.BlockSpec((_KD, _TOK), lambda j: (0, 0)),
            pl.BlockSpec((_VP, _TOK), lambda j: (0, 0)),
            pl.BlockSpec((1, _TOK), lambda j: (0, 0)),
        ],
        out_specs=pl.BlockSpec((_NB, _TOK), lambda j: (j, 0)),
        out_shape=jax.ShapeDtypeStruct((_NPAIR, _TOK), jnp.float32),
    )(acc2, vis3, m2, wemb, wvis, bias)


@jax.jit
def kernel(embeddings, visibility_scores, masks, W, b):
    emb_rows = embeddings.reshape(_NROW, _KD)
    mask_i32 = masks.astype(jnp.int32).reshape(_NROW)

    vis3 = jnp.pad(visibility_scores, ((0, 0), (0, 0), (0, 0), (0, _VP - _V)))
    vis3 = vis3.reshape(_NPAIR, _T, _VP)
    m2 = masks.astype(jnp.float32).reshape(_NPAIR, _T)
    wemb = W[:, :_KD].T
    wvis = jnp.pad(W[:, _KD:], ((0, 0), (0, _VP - _V))).T
    bias = b.reshape(1, _TOK)

    acc = _sc_reduce(emb_rows, mask_i32).reshape(_NPAIR, _KD)
    return _tc_proj(acc, vis3, m2, wemb, wvis, bias).reshape(_B, _N, _TOK)


# speedup vs baseline: 2.1026x; 2.1026x over previous
import functools

import jax
import jax.numpy as jnp
from jax.experimental import pallas as pl
from jax.experimental.pallas import tpu as pltpu

_B, _N, _T, _K, _D, _V, _TOK = 8, 256, 8, 7, 256, 7, 64
_KD = _K * _D
_VP = 8
_NB = 128


def _proj_kernel(emb_ref, vis_ref, m_ref, wemb_ref, wvis_ref, bias_ref, out_ref):
    m = m_ref[0]
    s = jnp.sum(m, axis=1, keepdims=True)
    scale = jnp.where(s > 0.0, 1.0 / jnp.maximum(s, 1.0), 0.0)
    w = m * scale

    e = emb_ref[0]
    ew = jnp.sum(e * w[:, :, None], axis=1)
    vis = vis_ref[0]
    vw = jnp.sum(vis * w[:, :, None], axis=1)

    acc = jax.lax.dot_general(ew, wemb_ref[...], (((1,), (0,)), ((), ())),
                              preferred_element_type=jnp.float32)
    acc = acc + jax.lax.dot_general(vw, wvis_ref[...], (((1,), (0,)), ((), ())),
                                    preferred_element_type=jnp.float32)
    any_m = (s > 0.0).astype(jnp.float32)
    out_ref[0] = acc + any_m * bias_ref[...]


@jax.jit
def kernel(embeddings, visibility_scores, masks, W, b):
    emb = embeddings.reshape(_B, _N, _T, _KD)
    vis = jnp.pad(visibility_scores, ((0, 0), (0, 0), (0, 0), (0, _VP - _V)))
    m = masks.astype(jnp.float32)
    wemb = W[:, :_KD].T
    wvis = jnp.pad(W[:, _KD:], ((0, 0), (0, _VP - _V))).T
    bias = b.reshape(1, _TOK)

    grid = (_B, _N // _NB)
    return pl.pallas_call(
        _proj_kernel,
        grid=grid,
        in_specs=[
            pl.BlockSpec((1, _NB, _T, _KD), lambda i, j: (i, j, 0, 0)),
            pl.BlockSpec((1, _NB, _T, _VP), lambda i, j: (i, j, 0, 0)),
            pl.BlockSpec((1, _NB, _T), lambda i, j: (i, j, 0)),
            pl.BlockSpec((_KD, _TOK), lambda i, j: (0, 0)),
            pl.BlockSpec((_VP, _TOK), lambda i, j: (0, 0)),
            pl.BlockSpec((1, _TOK), lambda i, j: (0, 0)),
        ],
        out_specs=pl.BlockSpec((1, _NB, _TOK), lambda i, j: (i, j, 0)),
        out_shape=jax.ShapeDtypeStruct((_B, _N, _TOK), jnp.float32),
    )(emb, vis, m, wemb, wvis, bias)


# speedup vs baseline: 2.2161x; 1.0540x over previous
import functools

import jax
import jax.numpy as jnp
from jax.experimental import pallas as pl
from jax.experimental.pallas import tpu as pltpu

_B, _N, _T, _K, _D, _V, _TOK = 8, 256, 8, 7, 256, 7, 64
_KD = _K * _D
_VP = 8
_NB = 256


def _proj_kernel(emb_ref, vis_ref, m_ref, wemb_ref, wvis_ref, bias_ref, out_ref):
    m = m_ref[0]
    s = jnp.sum(m, axis=1, keepdims=True)
    scale = jnp.where(s > 0.0, 1.0 / jnp.maximum(s, 1.0), 0.0)
    w = m * scale

    e = emb_ref[0]
    ew = jnp.sum(e * w[:, :, None], axis=1)
    vis = vis_ref[0]
    vw = jnp.sum(vis * w[:, :, None], axis=1)

    acc = jax.lax.dot_general(ew, wemb_ref[...], (((1,), (0,)), ((), ())),
                              preferred_element_type=jnp.float32)
    acc = acc + jax.lax.dot_general(vw, wvis_ref[...], (((1,), (0,)), ((), ())),
                                    preferred_element_type=jnp.float32)
    any_m = (s > 0.0).astype(jnp.float32)
    out_ref[0] = acc + any_m * bias_ref[...]


@jax.jit
def kernel(embeddings, visibility_scores, masks, W, b):
    emb = embeddings.reshape(_B, _N, _T, _KD)
    vis = jnp.pad(visibility_scores, ((0, 0), (0, 0), (0, 0), (0, _VP - _V)))
    m = masks.astype(jnp.float32)
    wemb = W[:, :_KD].T
    wvis = jnp.pad(W[:, _KD:], ((0, 0), (0, _VP - _V))).T
    bias = b.reshape(1, _TOK)

    grid = (_B, _N // _NB)
    return pl.pallas_call(
        _proj_kernel,
        grid=grid,
        in_specs=[
            pl.BlockSpec((1, _NB, _T, _KD), lambda i, j: (i, j, 0, 0)),
            pl.BlockSpec((1, _NB, _T, _VP), lambda i, j: (i, j, 0, 0)),
            pl.BlockSpec((1, _NB, _T), lambda i, j: (i, j, 0)),
            pl.BlockSpec((_KD, _TOK), lambda i, j: (0, 0)),
            pl.BlockSpec((_VP, _TOK), lambda i, j: (0, 0)),
            pl.BlockSpec((1, _TOK), lambda i, j: (0, 0)),
        ],
        out_specs=pl.BlockSpec((1, _NB, _TOK), lambda i, j: (i, j, 0)),
        out_shape=jax.ShapeDtypeStruct((_B, _N, _TOK), jnp.float32),
    )(emb, vis, m, wemb, wvis, bias)
